# baseline (speedup 1.0000x reference)
import functools

import jax
import jax.numpy as jnp
from jax import lax
from jax.experimental import pallas as pl
from jax.experimental.pallas import tpu as pltpu
from jax.experimental.pallas import tpu_sc as plsc

D_MODEL = 1024
LANES = 16
VECS_PER_ROW = D_MODEL // LANES
NUM_CORES = 2
NUM_SUBCORES = 16
NUM_WORKERS = NUM_CORES * NUM_SUBCORES
CHUNK = 32


def _build(batch):
    b_per_w = batch // NUM_WORKERS
    n_chunks = b_per_w // CHUNK
    mesh = plsc.VectorSubcoreMesh(core_axis_name="c", subcore_axis_name="s")

    buf_t = pltpu.VMEM((CHUNK, D_MODEL), jnp.float32)

    @functools.partial(
        pl.kernel,
        mesh=mesh,
        out_type=jax.ShapeDtypeStruct((batch, D_MODEL), jnp.float32),
        scratch_types=[
            pltpu.VMEM((n_chunks, CHUNK), jnp.int32),
            [buf_t, buf_t],
            buf_t,
            pltpu.SemaphoreType.DMA,
            pltpu.SemaphoreType.DMA,
        ],
    )
    def embed_sum(t0_hbm, t1_hbm, idx_hbm, out_hbm, idx_v, bufs_a, buf_b,
                  sem_g, sem_o):
        wid = lax.axis_index("s") * NUM_CORES + lax.axis_index("c")
        pltpu.sync_copy(idx_hbm.at[wid], idx_v)

        g0 = [None] * n_chunks
        g1 = [None] * n_chunks
        store_cp = [None, None]

        g0[0] = pltpu.async_copy(t0_hbm.at[idx_v.at[0]], bufs_a[0], sem_g)
        g1[0] = pltpu.async_copy(t1_hbm.at[idx_v.at[0]], buf_b, sem_g)
        for j in range(n_chunks):
            a = bufs_a[j % 2]
            if j + 1 < n_chunks:
                nxt = (j + 1) % 2
                if store_cp[nxt] is not None:
                    store_cp[nxt].wait()
                g0[j + 1] = pltpu.async_copy(
                    t0_hbm.at[idx_v.at[j + 1]], bufs_a[nxt], sem_g
                )
            g0[j].wait()
            g1[j].wait()

            @plsc.parallel_loop(0, CHUNK * VECS_PER_ROW, unroll=8)
            def _(i):
                r = i // VECS_PER_ROW
                c = (i % VECS_PER_ROW) * LANES
                plsc.addupdate(a.at[r, pl.ds(c, LANES)], buf_b[r, pl.ds(c, LANES)])

            if j + 1 < n_chunks:
                g1[j + 1] = pltpu.async_copy(
                    t1_hbm.at[idx_v.at[j + 1]], buf_b, sem_g
                )
            base = (wid * n_chunks + j) * CHUNK
            store_cp[j % 2] = pltpu.async_copy(
                a, out_hbm.at[pl.ds(base, CHUNK)], sem_o
            )
        for p in range(2):
            if store_cp[p] is not None:
                store_cp[p].wait()

    return embed_sum


@jax.jit
def kernel(tokens, table0, table1):
    b, s = tokens.shape
    batch = b * s
    idx = tokens.astype(jnp.int32).reshape(NUM_WORKERS, -1, CHUNK)
    out = _build(batch)(table0, table1, idx)
    return out.reshape(b, s, D_MODEL)


# speedup vs baseline: 1.1232x; 1.1232x over previous
import functools

import jax
import jax.numpy as jnp
from jax import lax
from jax.experimental import pallas as pl
from jax.experimental.pallas import tpu as pltpu
from jax.experimental.pallas import tpu_sc as plsc

D_MODEL = 1024
LANES = 16
VECS_PER_ROW = D_MODEL // LANES
NUM_CORES = 2
NUM_SUBCORES = 16
NUM_WORKERS = NUM_CORES * NUM_SUBCORES
CHUNK = 16
NBUF = 3


def _build(batch):
    b_per_w = batch // NUM_WORKERS
    n_chunks = b_per_w // CHUNK
    mesh = plsc.VectorSubcoreMesh(core_axis_name="c", subcore_axis_name="s")

    buf_t = pltpu.VMEM((CHUNK, D_MODEL), jnp.float32)

    @functools.partial(
        pl.kernel,
        mesh=mesh,
        out_type=jax.ShapeDtypeStruct((batch, D_MODEL), jnp.float32),
        scratch_types=[
            pltpu.VMEM((n_chunks, CHUNK), jnp.int32),
            [buf_t] * NBUF,
            [buf_t] * NBUF,
            pltpu.SemaphoreType.DMA,
            pltpu.SemaphoreType.DMA,
        ],
    )
    def embed_sum(t0_hbm, t1_hbm, idx_hbm, out_hbm, idx_v, bufs0, bufs1,
                  sem_g, sem_o):
        wid = lax.axis_index("s") * NUM_CORES + lax.axis_index("c")
        pltpu.sync_copy(idx_hbm.at[wid], idx_v)

        gather_cp = [None] * NBUF
        store_cp = [None] * NBUF

        def issue(j):
            p = j % NBUF
            if store_cp[p] is not None:
                store_cp[p].wait()
                store_cp[p] = None
            gather_cp[p] = (
                pltpu.async_copy(t0_hbm.at[idx_v.at[j]], bufs0[p], sem_g),
                pltpu.async_copy(t1_hbm.at[idx_v.at[j]], bufs1[p], sem_g),
            )

        for j in range(NBUF - 1):
            issue(j)
        for j in range(n_chunks):
            p = j % NBUF
            if j + NBUF - 1 < n_chunks:
                issue(j + NBUF - 1)
            c0, c1 = gather_cp[p]
            c0.wait()
            c1.wait()
            b0, b1 = bufs0[p], bufs1[p]

            @plsc.parallel_loop(0, CHUNK * VECS_PER_ROW, unroll=8)
            def _(i):
                r = i // VECS_PER_ROW
                c = (i % VECS_PER_ROW) * LANES
                plsc.addupdate(b0.at[r, pl.ds(c, LANES)], b1[r, pl.ds(c, LANES)])

            base = (wid * n_chunks + j) * CHUNK
            store_cp[p] = pltpu.async_copy(
                b0, out_hbm.at[pl.ds(base, CHUNK)], sem_o
            )
        for p in range(NBUF):
            if store_cp[p] is not None:
                store_cp[p].wait()

    return embed_sum


@jax.jit
def kernel(tokens, table0, table1):
    b, s = tokens.shape
    batch = b * s
    idx = tokens.astype(jnp.int32).reshape(NUM_WORKERS, -1, CHUNK)
    out = _build(batch)(table0, table1, idx)
    return out.reshape(b, s, D_MODEL)
